# Initial kernel scaffold; baseline (speedup 1.0000x reference)
import jax
import jax.numpy as jnp
from jax.experimental import pallas as pl


def kernel(node_features, e_ij, edge_index, W1, b1, g1, be1, W2, b2, g2, be2, W3, b3, g3, be3, W4, b4, g4, be4):
    raise NotImplementedError("write your pallas kernel here")


# speedup vs baseline: 4.5657x; 4.5657x over previous
import functools

import jax
import jax.numpy as jnp
from jax import lax
from jax.experimental import pallas as pl
from jax.experimental.pallas import tpu as pltpu
from jax.experimental.pallas import tpu_sc as plsc

N = 10000
K = 16
M = N * K
TN = 200
TM = TN * K
GRID = M // TM
CNT = float(M)
EPS = 1e-5

_SC_CHUNK = 200


def _sc_gather(nf_t, idx1, idx0):
    info = plsc.get_sparse_core_info()
    nc, ns = info.num_cores, info.num_subcores
    nw = nc * ns
    per_w = M // nw
    n_it = per_w // _SC_CHUNK
    mesh = plsc.VectorSubcoreMesh(core_axis_name="c", subcore_axis_name="s")

    @functools.partial(
        pl.kernel,
        mesh=mesh,
        out_type=(
            jax.ShapeDtypeStruct((M, 128), jnp.float32),
            jax.ShapeDtypeStruct((M, 128), jnp.float32),
        ),
        scratch_types=[
            pltpu.VMEM((_SC_CHUNK,), jnp.int32),
            pltpu.VMEM((_SC_CHUNK, 128), jnp.float32),
            pltpu.VMEM((_SC_CHUNK,), jnp.int32),
            pltpu.VMEM((_SC_CHUNK, 128), jnp.float32),
            pltpu.SemaphoreType.DMA,
            pltpu.SemaphoreType.DMA,
        ],
    )
    def k(nf_hbm, i1_hbm, i0_hbm, hi_hbm, hj_hbm,
          idx_a, rows_a, idx_b, rows_b, sem_a, sem_b):
        wid = lax.axis_index("s") * nc + lax.axis_index("c")
        base0 = wid * per_w

        def body(c, _):
            base = base0 + c * _SC_CHUNK
            pltpu.sync_copy(i1_hbm.at[pl.ds(base, _SC_CHUNK)], idx_a)
            cp_a = pltpu.async_copy(nf_hbm.at[idx_a], rows_a, sem_a)
            pltpu.sync_copy(i0_hbm.at[pl.ds(base, _SC_CHUNK)], idx_b)
            cp_b = pltpu.async_copy(nf_hbm.at[idx_b], rows_b, sem_b)
            cp_a.wait()
            pltpu.sync_copy(rows_a, hi_hbm.at[pl.ds(base, _SC_CHUNK)])
            cp_b.wait()
            pltpu.sync_copy(rows_b, hj_hbm.at[pl.ds(base, _SC_CHUNK)])
            return ()

        lax.fori_loop(0, n_it, body, (), unroll=False)

    return k(nf_t, idx1, idx0)


def _bn_affine(sq, g, be):
    mean = sq[0:1, :] / CNT
    var = sq[1:2, :] / CNT - mean * mean
    a = g * lax.rsqrt(var + EPS)
    c = be - mean * a
    return a, c


def _acc_stats(i, y, acc_s, acc_q, sq_ref):
    s = jnp.sum(y, axis=0, keepdims=True)
    q = jnp.sum(y * y, axis=0, keepdims=True)

    @pl.when(i == 0)
    def _():
        acc_s[...] = jnp.zeros_like(acc_s)
        acc_q[...] = jnp.zeros_like(acc_q)

    acc_s[...] += s
    acc_q[...] += q

    @pl.when(i == GRID - 1)
    def _():
        sq_ref[0:1, :] = acc_s[...]
        sq_ref[1:2, :] = acc_q[...]


def _stage1_body(e_ref, hi_ref, hj_ref, w1e, w1i, w1j,
                 y1_ref, sq_ref, acc_s, acc_q):
    i = pl.program_id(0)
    y = jnp.dot(e_ref[...], w1e[...], preferred_element_type=jnp.float32)
    y += jnp.dot(hi_ref[...], w1i[...], preferred_element_type=jnp.float32)
    y += jnp.dot(hj_ref[...], w1j[...], preferred_element_type=jnp.float32)
    y1_ref[...] = y
    _acc_stats(i, y, acc_s, acc_q, sq_ref)


def _stage2_body(y1_ref, sq1_ref, g1, be1, w2,
                 y2_ref, sq_ref, acc_s, acc_q):
    i = pl.program_id(0)
    a, c = _bn_affine(sq1_ref[...], g1[...], be1[...])
    e1 = jnp.maximum(y1_ref[...] * a + c, 0.0)
    y = jnp.dot(e1, w2[...], preferred_element_type=jnp.float32)
    y2_ref[...] = y
    _acc_stats(i, y, acc_s, acc_q, sq_ref)


def _stage3_body(y2_ref, sq2_ref, g2, be2, hi_ref, w3h, w3m,
                 e2_ref, y3_ref, sq_ref, acc_s, acc_q):
    i = pl.program_id(0)
    a, c = _bn_affine(sq2_ref[...], g2[...], be2[...])
    e2 = jnp.maximum(y2_ref[...] * a + c, 0.0)
    e2_ref[...] = e2
    m = jnp.sum(e2.reshape(TN, K, 128), axis=1)
    mm = jnp.dot(m, w3m[...], preferred_element_type=jnp.float32)
    y = jnp.dot(hi_ref[...], w3h[...], preferred_element_type=jnp.float32)
    y += jnp.broadcast_to(mm[:, None, :], (TN, K, 256)).reshape(TM, 256)
    y3_ref[...] = y
    _acc_stats(i, y, acc_s, acc_q, sq_ref)


def _stage4_body(y3_ref, sq3_ref, g3, be3, w4,
                 y4k0_ref, sq_ref, acc_s, acc_q):
    i = pl.program_id(0)
    a, c = _bn_affine(sq3_ref[...], g3[...], be3[...])
    n1 = jnp.maximum(y3_ref[...] * a + c, 0.0)
    y = jnp.dot(n1, w4[...], preferred_element_type=jnp.float32)
    y4k0_ref[...] = y.reshape(TN, K, 128)[:, 0:1, :].reshape(TN, 128)
    _acc_stats(i, y, acc_s, acc_q, sq_ref)


def _stage5_body(y4_ref, sq4_ref, g4, be4, out_ref):
    a, c = _bn_affine(sq4_ref[...], g4[...], be4[...])
    out_ref[...] = jnp.maximum(y4_ref[...] * a + c, 0.0)


def _row_spec(ch):
    return pl.BlockSpec((TM, ch), lambda i: (i, 0))


def _whole(shape):
    return pl.BlockSpec(shape, lambda i: tuple(0 for _ in shape))


def _sq_shape(ch):
    return jax.ShapeDtypeStruct((2, ch), jnp.float32)


def _scratch(ch):
    return [pltpu.VMEM((1, ch), jnp.float32),
            pltpu.VMEM((1, ch), jnp.float32)]


def _tc_chain(e_t, hi, hj, w1e, w1i, w1j, g1, be1, w2, g2, be2,
              w3h, w3m, g3, be3, w4, g4, be4):

    y1, sq1 = pl.pallas_call(
        _stage1_body,
        grid=(GRID,),
        in_specs=[_row_spec(16), _row_spec(128), _row_spec(128),
                  _whole((16, 256)), _whole((128, 256)), _whole((128, 256))],
        out_specs=[_row_spec(256), _whole((2, 256))],
        out_shape=[jax.ShapeDtypeStruct((M, 256), jnp.float32), _sq_shape(256)],
        scratch_shapes=_scratch(256),
    )(e_t, hi, hj, w1e, w1i, w1j)

    y2, sq2 = pl.pallas_call(
        _stage2_body,
        grid=(GRID,),
        in_specs=[_row_spec(256), _whole((2, 256)),
                  _whole((1, 256)), _whole((1, 256)), _whole((256, 128))],
        out_specs=[_row_spec(128), _whole((2, 128))],
        out_shape=[jax.ShapeDtypeStruct((M, 128), jnp.float32), _sq_shape(128)],
        scratch_shapes=_scratch(128),
    )(y1, sq1, g1, be1, w2)

    e2, y3, sq3 = pl.pallas_call(
        _stage3_body,
        grid=(GRID,),
        in_specs=[_row_spec(128), _whole((2, 128)),
                  _whole((1, 128)), _whole((1, 128)), _row_spec(128),
                  _whole((128, 256)), _whole((128, 256))],
        out_specs=[_row_spec(128), _row_spec(256), _whole((2, 256))],
        out_shape=[jax.ShapeDtypeStruct((M, 128), jnp.float32),
                   jax.ShapeDtypeStruct((M, 256), jnp.float32), _sq_shape(256)],
        scratch_shapes=_scratch(256),
    )(y2, sq2, g2, be2, hi, w3h, w3m)

    y4k0, sq4 = pl.pallas_call(
        _stage4_body,
        grid=(GRID,),
        in_specs=[_row_spec(256), _whole((2, 256)),
                  _whole((1, 256)), _whole((1, 256)), _whole((256, 128))],
        out_specs=[pl.BlockSpec((TN, 128), lambda i: (i, 0)),
                   _whole((2, 128))],
        out_shape=[jax.ShapeDtypeStruct((N, 128), jnp.float32), _sq_shape(128)],
        scratch_shapes=_scratch(128),
    )(y3, sq3, g3, be3, w4)

    h_out = pl.pallas_call(
        _stage5_body,
        in_specs=[pl.BlockSpec((N, 128), lambda: (0, 0)),
                  pl.BlockSpec((2, 128), lambda: (0, 0)),
                  pl.BlockSpec((1, 128), lambda: (0, 0)),
                  pl.BlockSpec((1, 128), lambda: (0, 0))],
        out_specs=pl.BlockSpec((N, 128), lambda: (0, 0)),
        out_shape=jax.ShapeDtypeStruct((N, 128), jnp.float32),
    )(y4k0, sq4, g4, be4)

    return e2, h_out


def kernel(node_features, e_ij, edge_index,
           W1, b1, g1, be1, W2, b2, g2, be2,
           W3, b3, g3, be3, W4, b4, g4, be4):
    del b1, b2, b3, b4
    nf_t = node_features[0, :, :, 0].T
    e_t = e_ij[0].transpose(1, 2, 0).reshape(M, 16)
    idx1 = edge_index[1, 0].reshape(M)
    idx0 = edge_index[0, 0].reshape(M)

    hi, hj = _sc_gather(nf_t, idx1, idx0)

    w1e = W1[:, :16].T
    w1i = W1[:, 16:144].T
    w1j = W1[:, 144:].T
    w3h = W3[:, :128].T
    w3m = W3[:, 128:].T
    r = lambda v: v.reshape(1, -1)

    e2, h_out = _tc_chain(e_t, hi, hj, w1e, w1i, w1j, r(g1), r(be1), W2.T,
                          r(g2), r(be2), w3h, w3m, r(g3), r(be3), W4.T,
                          r(g4), r(be4))

    e_ij_prima = e2.reshape(N, K, 128).transpose(2, 0, 1)[None]
    h_i_prima = h_out.T[None, :, :, None]
    return (h_i_prima, e_ij_prima, edge_index)
